# Initial kernel scaffold; baseline (speedup 1.0000x reference)
import jax
import jax.numpy as jnp
from jax.experimental import pallas as pl


def kernel(x, edge_index, t, msg, src, dst, memory_table, last_update, w_t, b_t, W_msg, b_msg, W_root, b_root, W_src, W_dst, b_h, W_out, b_out):
    raise NotImplementedError("write your pallas kernel here")


# speedup vs baseline: 9.3064x; 9.3064x over previous
import functools

import jax
import jax.numpy as jnp
from jax import lax
from jax.experimental import pallas as pl
from jax.experimental.pallas import tpu as pltpu
from jax.experimental.pallas import tpu_sc as plsc

_K = 80
_K0 = 400
_KQ = 64
_PW = 128


@functools.cache
def _make_a0(E, N):
    EPW = E // 32
    NCHUNK = EPW // _K0
    mesh = plsc.VectorSubcoreMesh(core_axis_name="c", subcore_axis_name="s")

    @functools.partial(
        pl.kernel,
        compiler_params=pltpu.CompilerParams(needs_layout_passes=False),
        out_type=jax.ShapeDtypeStruct((E,), jnp.float32),
        mesh=mesh,
        scratch_types=[
            pltpu.VMEM((_K0,), jnp.int32),
            pltpu.VMEM((_K0,), jnp.float32),
            pltpu.VMEM((_K0,), jnp.float32),
            pltpu.VMEM((N,), jnp.float32),
        ],
    )
    def a0(src_hbm, t_hbm, lu_hbm, rel_out, sidx, tbuf, rbuf, lut):
        c = lax.axis_index("c")
        s = lax.axis_index("s")
        wid = s * 2 + c
        pltpu.sync_copy(lu_hbm, lut)
        base0 = wid * EPW

        def body(i, carry):
            base = base0 + i * _K0
            pltpu.sync_copy(src_hbm.at[pl.ds(base, _K0)], sidx)
            pltpu.sync_copy(t_hbm.at[pl.ds(base, _K0)], tbuf)
            for j in range(_K0 // 16):
                sv = sidx[pl.ds(j * 16, 16)]
                lus = plsc.load_gather(lut, [sv])
                rbuf[pl.ds(j * 16, 16)] = tbuf[pl.ds(j * 16, 16)] - lus
            pltpu.sync_copy(rbuf, rel_out.at[pl.ds(base, _K0)])
            return carry

        lax.fori_loop(0, NCHUNK, body, 0)

    return a0


def _stage_r(rel_t, w_row, b_row, msg):
    E, MD = msg.shape
    TD = w_row.shape[1]
    R = 2000 if E % 2000 == 0 else E
    G = E // R

    def rk(rel_r, w_r, b_r, msg_r, out_r):
        renc = jnp.cos(rel_r[...] * w_r[...] + b_r[...])
        ones = jnp.ones((R, 1), jnp.float32)
        pad1 = jnp.zeros((R, 15), jnp.float32)
        pad2 = jnp.zeros((R, _PW - TD - 16 - MD), jnp.float32)
        out_r[...] = jnp.concatenate(
            [renc, ones, pad1, msg_r[...], pad2], axis=1)

    return pl.pallas_call(
        rk,
        grid=(G,),
        in_specs=[
            pl.BlockSpec((R, 1), lambda i: (i, 0)),
            pl.BlockSpec((1, TD), lambda i: (0, 0)),
            pl.BlockSpec((1, TD), lambda i: (0, 0)),
            pl.BlockSpec((R, MD), lambda i: (i, 0)),
        ],
        out_specs=pl.BlockSpec((R, _PW), lambda i: (i, 0)),
        out_shape=jax.ShapeDtypeStruct((E, _PW), jnp.float32),
    )(rel_t.reshape(E, 1), w_row, b_row, msg)


@functools.cache
def _make_az(E, N, D):
    EPW = E // 32
    NCHUNK = EPW // _K
    NP = ((N + 127) // 128) * 128
    RPT = NP // 16
    mesh = plsc.VectorSubcoreMesh(core_axis_name="c", subcore_axis_name="s")

    @functools.partial(
        pl.kernel,
        compiler_params=pltpu.CompilerParams(needs_layout_passes=False),
        out_type=jax.ShapeDtypeStruct((2 * NP, D), jnp.float32),
        mesh=mesh,
        scratch_types=[
            pltpu.VMEM((_K,), jnp.int32),
            pltpu.VMEM((_K,), jnp.int32),
            pltpu.VMEM((_K, D), jnp.float32),
            pltpu.VMEM_SHARED((NP, D), jnp.float32),
            pltpu.SemaphoreType.DMA,
        ],
    )
    def az(src_hbm, dst_hbm, mem_hbm, zeros_hbm, out_hbm, sidx, didx, rows,
           acc, sem):
        c = lax.axis_index("c")
        s = lax.axis_index("s")
        wid = s * 2 + c
        pltpu.sync_copy(zeros_hbm, acc.at[pl.ds(s * RPT, RPT)])
        plsc.subcore_barrier()
        base0 = wid * EPW

        def body(i, carry):
            base = base0 + i * _K
            pltpu.sync_copy(src_hbm.at[pl.ds(base, _K)], sidx)
            pltpu.sync_copy(dst_hbm.at[pl.ds(base, _K)], didx)
            pltpu.async_copy(mem_hbm.at[sidx], rows, sem).wait()
            pltpu.sync_copy(rows, acc.at[didx], add=True)
            return carry

        lax.fori_loop(0, NCHUNK, body, 0)
        plsc.subcore_barrier()
        pltpu.sync_copy(acc.at[pl.ds(s * RPT, RPT)],
                        out_hbm.at[pl.ds(c * NP + s * RPT, RPT)])

    return az


@functools.cache
def _make_a2(E, N):
    EPW = E // 32
    NCHUNK = EPW // _K
    NP = ((N + 127) // 128) * 128
    RPT = NP // 16
    mesh = plsc.VectorSubcoreMesh(core_axis_name="c", subcore_axis_name="s")

    @functools.partial(
        pl.kernel,
        compiler_params=pltpu.CompilerParams(needs_layout_passes=False),
        out_type=jax.ShapeDtypeStruct((2 * NP, _PW), jnp.float32),
        mesh=mesh,
        scratch_types=[
            pltpu.VMEM((_K,), jnp.int32),
            pltpu.VMEM((_K,), jnp.int32),
            pltpu.VMEM((_K, _PW), jnp.float32),
            pltpu.VMEM((_K, _PW), jnp.float32),
            pltpu.VMEM_SHARED((NP, _PW), jnp.float32),
        ],
    )
    def a2(src_hbm, dst_hbm, pack_hbm, zp_hbm, ones_hbm, p_out,
           sidx, didx, prows, tones, acc_p):
        c = lax.axis_index("c")
        s = lax.axis_index("s")
        wid = s * 2 + c
        pltpu.sync_copy(zp_hbm, acc_p.at[pl.ds(s * RPT, RPT)])
        pltpu.sync_copy(ones_hbm, tones)
        plsc.subcore_barrier()
        base0 = wid * EPW

        def body(i, carry):
            base = base0 + i * _K
            pltpu.sync_copy(src_hbm.at[pl.ds(base, _K)], sidx)
            pltpu.sync_copy(dst_hbm.at[pl.ds(base, _K)], didx)
            pltpu.sync_copy(pack_hbm.at[pl.ds(base, _K)], prows)
            pltpu.sync_copy(prows, acc_p.at[didx], add=True)
            pltpu.sync_copy(tones, acc_p.at[sidx], add=True)
            return carry

        lax.fori_loop(0, NCHUNK, body, 0)
        plsc.subcore_barrier()
        pltpu.sync_copy(acc_p.at[pl.ds(s * RPT, RPT)],
                        p_out.at[pl.ds(c * NP + s * RPT, RPT)])

    return a2


def _stage_b(az0, az1, p0, p1, mem, W1, W2, W_root, W_src, W_dst,
             b_root, b_h):
    N, D = mem.shape
    R = 400 if N % 400 == 0 else N
    G = N // R
    TD = 64

    def bk(az0_r, az1_r, p0_r, p1_r, mem_r, w1_r, w2_r, wr_r,
           ws_r, wd_r, brt_r, bh_r, zs_o, zd_o, mk_o, v0_o, vmin_s):
        i = pl.program_id(0)
        azv = az0_r[...] + az1_r[...]
        pv = p0_r[...] + p1_r[...]
        tv = pv[:, 96:97]
        deg = pv[:, TD:TD + 1]
        inv = 1.0 / jnp.maximum(deg, 1.0)
        agg = (jnp.dot(azv * inv, w1_r[...], preferred_element_type=jnp.float32)
               + jnp.dot(pv * inv, w2_r[...], preferred_element_type=jnp.float32))
        zo = jnp.maximum(
            jnp.dot(mem_r[...], wr_r[...], preferred_element_type=jnp.float32)
            + brt_r[...] + agg, 0.0)
        zs_o[...] = jnp.dot(zo, ws_r[...],
                            preferred_element_type=jnp.float32) + bh_r[...]
        zd_o[...] = jnp.dot(zo, wd_r[...],
                            preferred_element_type=jnp.float32)
        maskc = (tv[:, 0:1] + deg) > 0.0
        mk_o[...] = maskc.astype(jnp.float32)
        ids = lax.broadcasted_iota(jnp.int32, (R, 1), 0) + i * R
        lmin = jnp.min(jnp.where(maskc, ids, N))

        @pl.when(i == 0)
        def _():
            vmin_s[0] = N

        vmin_s[0] = jnp.minimum(vmin_s[0], lmin)

        @pl.when(i == G - 1)
        def _():
            v0_o[0, 0] = vmin_s[0]

    row = lambda i: (i, 0)
    fix = lambda i: (0, 0)
    return pl.pallas_call(
        bk,
        grid=(G,),
        in_specs=[
            pl.BlockSpec((R, D), row), pl.BlockSpec((R, D), row),
            pl.BlockSpec((R, _PW), row), pl.BlockSpec((R, _PW), row),
            pl.BlockSpec((R, D), row),
            pl.BlockSpec((D, D), fix), pl.BlockSpec((_PW, D), fix),
            pl.BlockSpec((D, D), fix), pl.BlockSpec((D, D), fix),
            pl.BlockSpec((D, D), fix),
            pl.BlockSpec((1, D), fix), pl.BlockSpec((1, D), fix),
        ],
        out_specs=[
            pl.BlockSpec((R, D), row), pl.BlockSpec((R, D), row),
            pl.BlockSpec((R, 1), row),
            pl.BlockSpec(memory_space=pltpu.SMEM),
        ],
        out_shape=[
            jax.ShapeDtypeStruct((N, D), jnp.float32),
            jax.ShapeDtypeStruct((N, D), jnp.float32),
            jax.ShapeDtypeStruct((N, 1), jnp.float32),
            jax.ShapeDtypeStruct((1, 1), jnp.int32),
        ],
        scratch_shapes=[pltpu.SMEM((1,), jnp.int32)],
    )(az0, az1, p0, p1, mem, W1, W2, W_root, W_src, W_dst,
      b_root.reshape(1, D), b_h.reshape(1, D))


@functools.cache
def _make_c(N, D, B):
    QPW = B // 16
    NQC = QPW // _KQ
    mesh = plsc.VectorSubcoreMesh(core_axis_name="c", subcore_axis_name="s")

    @functools.partial(
        pl.kernel,
        compiler_params=pltpu.CompilerParams(needs_layout_passes=False),
        out_type=jax.ShapeDtypeStruct((2 * B, D), jnp.float32),
        mesh=mesh,
        scratch_types=[
            pltpu.VMEM((_KQ,), jnp.int32),
            pltpu.VMEM((_KQ,), jnp.int32),
            pltpu.VMEM((_KQ, D), jnp.float32),
            pltpu.VMEM((N,), jnp.float32),
            pltpu.VMEM((16,), jnp.int32),
            pltpu.SemaphoreType.DMA,
        ],
    )
    def ck(qcat_hbm, zcat_hbm, mask_hbm, v0_hbm, g_out, qidx, effb, rows,
           mv, v0v, sem):
        c = lax.axis_index("c")
        s = lax.axis_index("s")
        pltpu.sync_copy(mask_hbm, mv)
        pltpu.sync_copy(v0_hbm, v0v)
        v0vec = v0v[pl.ds(0, 16)]
        off = jnp.full((16,), 0, jnp.int32) + c * N

        def body(i, carry):
            base = c * B + s * QPW + i * _KQ
            pltpu.sync_copy(qcat_hbm.at[pl.ds(base, _KQ)], qidx)
            for j in range(_KQ // 16):
                qv = qidx[pl.ds(j * 16, 16)]
                m = plsc.load_gather(mv, [qv])
                effb[pl.ds(j * 16, 16)] = (
                    jnp.where(m > 0.5, qv, v0vec) + off)
            pltpu.async_copy(zcat_hbm.at[effb], rows, sem).wait()
            pltpu.sync_copy(rows, g_out.at[pl.ds(base, _KQ)])
            return carry

        lax.fori_loop(0, NQC, body, 0)

    return ck


def _stage_d(gs, gd, w_row, b_out):
    B, D = gs.shape
    R = 1024 if B % 1024 == 0 else B
    G = B // R

    def dk(gs_r, gd_r, w_r, bo_r, out_r):
        h = jnp.maximum(gs_r[...] + gd_r[...], 0.0)
        out_r[...] = jnp.sum(h * w_r[...], axis=1, keepdims=True) + bo_r[...]

    return pl.pallas_call(
        dk,
        grid=(G,),
        in_specs=[
            pl.BlockSpec((R, D), lambda i: (i, 0)),
            pl.BlockSpec((R, D), lambda i: (i, 0)),
            pl.BlockSpec((1, D), lambda i: (0, 0)),
            pl.BlockSpec((1, 1), lambda i: (0, 0)),
        ],
        out_specs=pl.BlockSpec((R, 1), lambda i: (i, 0)),
        out_shape=jax.ShapeDtypeStruct((B, 1), jnp.float32),
    )(gs, gd, w_row, b_out)


def kernel(x, edge_index, t, msg, src, dst, memory_table, last_update,
           w_t, b_t, W_msg, b_msg, W_root, b_root, W_src, W_dst, b_h,
           W_out, b_out):
    N, D = memory_table.shape
    E = t.shape[0]
    B = src.shape[0]
    TD = w_t.shape[0]
    MD = msg.shape[1]
    NP = ((N + 127) // 128) * 128
    RPT = NP // 16
    src_e = edge_index[0]
    dst_e = edge_index[1]

    zeros_d = jnp.zeros((RPT, D), jnp.float32)
    zeros_p = jnp.zeros((RPT, _PW), jnp.float32)
    tmark = jnp.zeros((_K, _PW), jnp.float32).at[:, 96].set(1.0)

    az = _make_az(E, N, D)(src_e, dst_e, memory_table, zeros_d)
    rel_t = _make_a0(E, N)(src_e, t, last_update)
    packed = _stage_r(rel_t, w_t.reshape(1, TD), b_t.reshape(1, TD), msg)
    p_p = _make_a2(E, N)(src_e, dst_e, packed, zeros_p, tmark)

    W2 = jnp.concatenate(
        [W_msg[D:D + TD], b_msg[None, :], jnp.zeros((15, D), jnp.float32),
         W_msg[D + TD:], jnp.zeros((_PW - TD - 16 - MD, D), jnp.float32)],
        axis=0)
    zs, zd, maskv, v0 = _stage_b(
        az[:N], az[NP:NP + N], p_p[:N], p_p[NP:NP + N],
        memory_table, W_msg[:D], W2, W_root, W_src, W_dst,
        b_root, b_h)

    zcat = jnp.concatenate([zs, zd], axis=0)
    qcat = jnp.concatenate([src, dst], axis=0)
    v16 = jnp.broadcast_to(v0.reshape(1), (16,))
    gcat = _make_c(N, D, B)(qcat, zcat, maskv.reshape(N), v16)

    return _stage_d(gcat[:B], gcat[B:], W_out.reshape(1, D),
                    b_out.reshape(1, 1))


# speedup vs baseline: 10.1887x; 1.0948x over previous
import functools

import jax
import jax.numpy as jnp
from jax import lax
from jax.experimental import pallas as pl
from jax.experimental.pallas import tpu as pltpu
from jax.experimental.pallas import tpu_sc as plsc

_K = 80
_K0 = 400
_KQ = 64
_PW = 128


@functools.cache
def _make_a0(E, N):
    EPW = E // 32
    NCHUNK = EPW // _K0
    mesh = plsc.VectorSubcoreMesh(core_axis_name="c", subcore_axis_name="s")

    @functools.partial(
        pl.kernel,
        compiler_params=pltpu.CompilerParams(needs_layout_passes=False),
        out_type=jax.ShapeDtypeStruct((E,), jnp.float32),
        mesh=mesh,
        scratch_types=[
            pltpu.VMEM((_K0,), jnp.int32),
            pltpu.VMEM((_K0,), jnp.float32),
            pltpu.VMEM((_K0,), jnp.float32),
            pltpu.VMEM((N,), jnp.float32),
        ],
    )
    def a0(src_hbm, t_hbm, lu_hbm, rel_out, sidx, tbuf, rbuf, lut):
        c = lax.axis_index("c")
        s = lax.axis_index("s")
        wid = s * 2 + c
        pltpu.sync_copy(lu_hbm, lut)
        base0 = wid * EPW

        def body(i, carry):
            base = base0 + i * _K0
            pltpu.sync_copy(src_hbm.at[pl.ds(base, _K0)], sidx)
            pltpu.sync_copy(t_hbm.at[pl.ds(base, _K0)], tbuf)
            for j in range(_K0 // 16):
                sv = sidx[pl.ds(j * 16, 16)]
                lus = plsc.load_gather(lut, [sv])
                rbuf[pl.ds(j * 16, 16)] = tbuf[pl.ds(j * 16, 16)] - lus
            pltpu.sync_copy(rbuf, rel_out.at[pl.ds(base, _K0)])
            return carry

        lax.fori_loop(0, NCHUNK, body, 0)

    return a0


def _stage_r(rel_t, w_row, b_row, msg):
    E, MD = msg.shape
    TD = w_row.shape[1]
    R = 2000 if E % 2000 == 0 else E
    G = E // R

    def rk(rel_r, w_r, b_r, msg_r, out_r):
        renc = jnp.cos(rel_r[...] * w_r[...] + b_r[...])
        ones = jnp.ones((R, 1), jnp.float32)
        pad1 = jnp.zeros((R, 15), jnp.float32)
        pad2 = jnp.zeros((R, _PW - TD - 16 - MD), jnp.float32)
        out_r[...] = jnp.concatenate(
            [renc, ones, pad1, msg_r[...], pad2], axis=1)

    return pl.pallas_call(
        rk,
        grid=(G,),
        in_specs=[
            pl.BlockSpec((R, 1), lambda i: (i, 0)),
            pl.BlockSpec((1, TD), lambda i: (0, 0)),
            pl.BlockSpec((1, TD), lambda i: (0, 0)),
            pl.BlockSpec((R, MD), lambda i: (i, 0)),
        ],
        out_specs=pl.BlockSpec((R, _PW), lambda i: (i, 0)),
        out_shape=jax.ShapeDtypeStruct((E, _PW), jnp.float32),
    )(rel_t.reshape(E, 1), w_row, b_row, msg)


@functools.cache
def _make_az(E, N, D):
    EPW = E // 32
    NCHUNK = EPW // _K
    NP = ((N + 127) // 128) * 128
    RPT = NP // 16
    mesh = plsc.VectorSubcoreMesh(core_axis_name="c", subcore_axis_name="s")

    @functools.partial(
        pl.kernel,
        compiler_params=pltpu.CompilerParams(needs_layout_passes=False),
        out_type=jax.ShapeDtypeStruct((2 * NP, D), jnp.float32),
        mesh=mesh,
        scratch_types=[
            pltpu.VMEM((_K,), jnp.int32),
            pltpu.VMEM((_K,), jnp.int32),
            pltpu.VMEM((_K, D), jnp.float32),
            pltpu.VMEM((_K,), jnp.int32),
            pltpu.VMEM((_K,), jnp.int32),
            pltpu.VMEM((_K, D), jnp.float32),
            pltpu.VMEM_SHARED((NP, D), jnp.float32),
            pltpu.SemaphoreType.DMA,
            pltpu.SemaphoreType.DMA,
        ],
    )
    def az(src_hbm, dst_hbm, mem_hbm, zeros_hbm, out_hbm, sidx, didx, rows,
           sidx2, didx2, rows2, acc, sem, sem2):
        c = lax.axis_index("c")
        s = lax.axis_index("s")
        wid = s * 2 + c
        pltpu.sync_copy(zeros_hbm, acc.at[pl.ds(s * RPT, RPT)])
        plsc.subcore_barrier()
        base0 = wid * EPW

        def body(i, carry):
            base = base0 + 2 * i * _K
            pltpu.sync_copy(src_hbm.at[pl.ds(base, _K)], sidx)
            pltpu.sync_copy(dst_hbm.at[pl.ds(base, _K)], didx)
            cp1 = pltpu.async_copy(mem_hbm.at[sidx], rows, sem)
            pltpu.sync_copy(src_hbm.at[pl.ds(base + _K, _K)], sidx2)
            pltpu.sync_copy(dst_hbm.at[pl.ds(base + _K, _K)], didx2)
            cp2 = pltpu.async_copy(mem_hbm.at[sidx2], rows2, sem2)
            cp1.wait()
            pltpu.sync_copy(rows, acc.at[didx], add=True)
            cp2.wait()
            pltpu.sync_copy(rows2, acc.at[didx2], add=True)
            return carry

        lax.fori_loop(0, NCHUNK // 2, body, 0)
        if NCHUNK % 2:
            base = base0 + (NCHUNK - 1) * _K
            pltpu.sync_copy(src_hbm.at[pl.ds(base, _K)], sidx)
            pltpu.sync_copy(dst_hbm.at[pl.ds(base, _K)], didx)
            pltpu.async_copy(mem_hbm.at[sidx], rows, sem).wait()
            pltpu.sync_copy(rows, acc.at[didx], add=True)
        plsc.subcore_barrier()
        pltpu.sync_copy(acc.at[pl.ds(s * RPT, RPT)],
                        out_hbm.at[pl.ds(c * NP + s * RPT, RPT)])

    return az


@functools.cache
def _make_a2(E, N):
    EPW = E // 32
    NCHUNK = EPW // _K
    NP = ((N + 127) // 128) * 128
    RPT = NP // 16
    mesh = plsc.VectorSubcoreMesh(core_axis_name="c", subcore_axis_name="s")

    @functools.partial(
        pl.kernel,
        compiler_params=pltpu.CompilerParams(needs_layout_passes=False),
        out_type=jax.ShapeDtypeStruct((2 * NP, _PW), jnp.float32),
        mesh=mesh,
        scratch_types=[
            pltpu.VMEM((_K,), jnp.int32),
            pltpu.VMEM((_K,), jnp.int32),
            pltpu.VMEM((_K, _PW), jnp.float32),
            pltpu.VMEM((_K,), jnp.int32),
            pltpu.VMEM((_K,), jnp.int32),
            pltpu.VMEM((_K, _PW), jnp.float32),
            pltpu.VMEM((_K, _PW), jnp.float32),
            pltpu.VMEM_SHARED((NP, _PW), jnp.float32),
            pltpu.SemaphoreType.DMA,
            pltpu.SemaphoreType.DMA,
        ],
    )
    def a2(src_hbm, dst_hbm, pack_hbm, zp_hbm, ones_hbm, p_out,
           sidx, didx, prows, sidx2, didx2, prows2, tones, acc_p,
           sem, sem2):
        c = lax.axis_index("c")
        s = lax.axis_index("s")
        wid = s * 2 + c
        pltpu.sync_copy(zp_hbm, acc_p.at[pl.ds(s * RPT, RPT)])
        pltpu.sync_copy(ones_hbm, tones)
        plsc.subcore_barrier()
        base0 = wid * EPW

        def body(i, carry):
            base = base0 + 2 * i * _K
            pltpu.sync_copy(src_hbm.at[pl.ds(base, _K)], sidx)
            pltpu.sync_copy(dst_hbm.at[pl.ds(base, _K)], didx)
            cp1 = pltpu.async_copy(pack_hbm.at[pl.ds(base, _K)], prows, sem)
            pltpu.sync_copy(src_hbm.at[pl.ds(base + _K, _K)], sidx2)
            pltpu.sync_copy(dst_hbm.at[pl.ds(base + _K, _K)], didx2)
            cp2 = pltpu.async_copy(pack_hbm.at[pl.ds(base + _K, _K)],
                                   prows2, sem2)
            cp1.wait()
            pltpu.sync_copy(prows, acc_p.at[didx], add=True)
            pltpu.sync_copy(tones, acc_p.at[sidx], add=True)
            cp2.wait()
            pltpu.sync_copy(prows2, acc_p.at[didx2], add=True)
            pltpu.sync_copy(tones, acc_p.at[sidx2], add=True)
            return carry

        lax.fori_loop(0, NCHUNK // 2, body, 0)
        if NCHUNK % 2:
            base = base0 + (NCHUNK - 1) * _K
            pltpu.sync_copy(src_hbm.at[pl.ds(base, _K)], sidx)
            pltpu.sync_copy(dst_hbm.at[pl.ds(base, _K)], didx)
            pltpu.sync_copy(pack_hbm.at[pl.ds(base, _K)], prows)
            pltpu.sync_copy(prows, acc_p.at[didx], add=True)
            pltpu.sync_copy(tones, acc_p.at[sidx], add=True)
        plsc.subcore_barrier()
        pltpu.sync_copy(acc_p.at[pl.ds(s * RPT, RPT)],
                        p_out.at[pl.ds(c * NP + s * RPT, RPT)])

    return a2


def _stage_b(az0, az1, p0, p1, mem, W1, W2, W_root, W_src, W_dst,
             b_root, b_h):
    N, D = mem.shape
    R = 400 if N % 400 == 0 else N
    G = N // R
    TD = 64

    def bk(az0_r, az1_r, p0_r, p1_r, mem_r, w1_r, w2_r, wr_r,
           ws_r, wd_r, brt_r, bh_r, zs_o, zd_o, mk_o, v0_o, vmin_s):
        i = pl.program_id(0)
        azv = az0_r[...] + az1_r[...]
        pv = p0_r[...] + p1_r[...]
        tv = pv[:, 96:97]
        deg = pv[:, TD:TD + 1]
        inv = 1.0 / jnp.maximum(deg, 1.0)
        agg = (jnp.dot(azv * inv, w1_r[...], preferred_element_type=jnp.float32)
               + jnp.dot(pv * inv, w2_r[...], preferred_element_type=jnp.float32))
        zo = jnp.maximum(
            jnp.dot(mem_r[...], wr_r[...], preferred_element_type=jnp.float32)
            + brt_r[...] + agg, 0.0)
        zs_o[...] = jnp.dot(zo, ws_r[...],
                            preferred_element_type=jnp.float32) + bh_r[...]
        zd_o[...] = jnp.dot(zo, wd_r[...],
                            preferred_element_type=jnp.float32)
        maskc = (tv[:, 0:1] + deg) > 0.0
        mk_o[...] = maskc.astype(jnp.float32)
        ids = lax.broadcasted_iota(jnp.int32, (R, 1), 0) + i * R
        lmin = jnp.min(jnp.where(maskc, ids, N))

        @pl.when(i == 0)
        def _():
            vmin_s[0] = N

        vmin_s[0] = jnp.minimum(vmin_s[0], lmin)

        @pl.when(i == G - 1)
        def _():
            v0_o[0, 0] = vmin_s[0]

    row = lambda i: (i, 0)
    fix = lambda i: (0, 0)
    return pl.pallas_call(
        bk,
        grid=(G,),
        in_specs=[
            pl.BlockSpec((R, D), row), pl.BlockSpec((R, D), row),
            pl.BlockSpec((R, _PW), row), pl.BlockSpec((R, _PW), row),
            pl.BlockSpec((R, D), row),
            pl.BlockSpec((D, D), fix), pl.BlockSpec((_PW, D), fix),
            pl.BlockSpec((D, D), fix), pl.BlockSpec((D, D), fix),
            pl.BlockSpec((D, D), fix),
            pl.BlockSpec((1, D), fix), pl.BlockSpec((1, D), fix),
        ],
        out_specs=[
            pl.BlockSpec((R, D), row), pl.BlockSpec((R, D), row),
            pl.BlockSpec((R, 1), row),
            pl.BlockSpec(memory_space=pltpu.SMEM),
        ],
        out_shape=[
            jax.ShapeDtypeStruct((N, D), jnp.float32),
            jax.ShapeDtypeStruct((N, D), jnp.float32),
            jax.ShapeDtypeStruct((N, 1), jnp.float32),
            jax.ShapeDtypeStruct((1, 1), jnp.int32),
        ],
        scratch_shapes=[pltpu.SMEM((1,), jnp.int32)],
    )(az0, az1, p0, p1, mem, W1, W2, W_root, W_src, W_dst,
      b_root.reshape(1, D), b_h.reshape(1, D))


@functools.cache
def _make_c(N, D, B):
    QPW = B // 16
    NQC = QPW // _KQ
    mesh = plsc.VectorSubcoreMesh(core_axis_name="c", subcore_axis_name="s")

    @functools.partial(
        pl.kernel,
        compiler_params=pltpu.CompilerParams(needs_layout_passes=False),
        out_type=jax.ShapeDtypeStruct((2 * B, D), jnp.float32),
        mesh=mesh,
        scratch_types=[
            pltpu.VMEM((_KQ,), jnp.int32),
            pltpu.VMEM((_KQ,), jnp.int32),
            pltpu.VMEM((_KQ, D), jnp.float32),
            pltpu.VMEM((N,), jnp.float32),
            pltpu.VMEM((16,), jnp.int32),
            pltpu.SemaphoreType.DMA,
        ],
    )
    def ck(qcat_hbm, zcat_hbm, mask_hbm, v0_hbm, g_out, qidx, effb, rows,
           mv, v0v, sem):
        c = lax.axis_index("c")
        s = lax.axis_index("s")
        pltpu.sync_copy(mask_hbm, mv)
        pltpu.sync_copy(v0_hbm, v0v)
        v0vec = v0v[pl.ds(0, 16)]
        off = jnp.full((16,), 0, jnp.int32) + c * N

        def body(i, carry):
            base = c * B + s * QPW + i * _KQ
            pltpu.sync_copy(qcat_hbm.at[pl.ds(base, _KQ)], qidx)
            for j in range(_KQ // 16):
                qv = qidx[pl.ds(j * 16, 16)]
                m = plsc.load_gather(mv, [qv])
                effb[pl.ds(j * 16, 16)] = (
                    jnp.where(m > 0.5, qv, v0vec) + off)
            pltpu.async_copy(zcat_hbm.at[effb], rows, sem).wait()
            pltpu.sync_copy(rows, g_out.at[pl.ds(base, _KQ)])
            return carry

        lax.fori_loop(0, NQC, body, 0)

    return ck


def _stage_d(gs, gd, w_row, b_out):
    B, D = gs.shape
    R = 1024 if B % 1024 == 0 else B
    G = B // R

    def dk(gs_r, gd_r, w_r, bo_r, out_r):
        h = jnp.maximum(gs_r[...] + gd_r[...], 0.0)
        out_r[...] = jnp.sum(h * w_r[...], axis=1, keepdims=True) + bo_r[...]

    return pl.pallas_call(
        dk,
        grid=(G,),
        in_specs=[
            pl.BlockSpec((R, D), lambda i: (i, 0)),
            pl.BlockSpec((R, D), lambda i: (i, 0)),
            pl.BlockSpec((1, D), lambda i: (0, 0)),
            pl.BlockSpec((1, 1), lambda i: (0, 0)),
        ],
        out_specs=pl.BlockSpec((R, 1), lambda i: (i, 0)),
        out_shape=jax.ShapeDtypeStruct((B, 1), jnp.float32),
    )(gs, gd, w_row, b_out)


def kernel(x, edge_index, t, msg, src, dst, memory_table, last_update,
           w_t, b_t, W_msg, b_msg, W_root, b_root, W_src, W_dst, b_h,
           W_out, b_out):
    N, D = memory_table.shape
    E = t.shape[0]
    B = src.shape[0]
    TD = w_t.shape[0]
    MD = msg.shape[1]
    NP = ((N + 127) // 128) * 128
    RPT = NP // 16
    src_e = edge_index[0]
    dst_e = edge_index[1]

    zeros_d = jnp.zeros((RPT, D), jnp.float32)
    zeros_p = jnp.zeros((RPT, _PW), jnp.float32)
    tmark = jnp.zeros((_K, _PW), jnp.float32).at[:, 96].set(1.0)

    az = _make_az(E, N, D)(src_e, dst_e, memory_table, zeros_d)
    rel_t = _make_a0(E, N)(src_e, t, last_update)
    packed = _stage_r(rel_t, w_t.reshape(1, TD), b_t.reshape(1, TD), msg)
    p_p = _make_a2(E, N)(src_e, dst_e, packed, zeros_p, tmark)

    W2 = jnp.concatenate(
        [W_msg[D:D + TD], b_msg[None, :], jnp.zeros((15, D), jnp.float32),
         W_msg[D + TD:], jnp.zeros((_PW - TD - 16 - MD, D), jnp.float32)],
        axis=0)
    zs, zd, maskv, v0 = _stage_b(
        az[:N], az[NP:NP + N], p_p[:N], p_p[NP:NP + N],
        memory_table, W_msg[:D], W2, W_root, W_src, W_dst,
        b_root, b_h)

    zcat = jnp.concatenate([zs, zd], axis=0)
    qcat = jnp.concatenate([src, dst], axis=0)
    v16 = jnp.broadcast_to(v0.reshape(1), (16,))
    gcat = _make_c(N, D, B)(qcat, zcat, maskv.reshape(N), v16)

    return _stage_d(gcat[:B], gcat[B:], W_out.reshape(1, D),
                    b_out.reshape(1, 1))
